# baseline (speedup 1.0000x reference)
import functools

import jax
import jax.numpy as jnp
from jax import lax
from jax.experimental import pallas as pl
from jax.experimental.pallas import tpu as pltpu
from jax.experimental.pallas import tpu_sc as plsc

B, J, O, D = 1024, 50, 20, 128
NC, NS, L = 2, 16, 16
NW = NC * NS
BPW = B // NW
ROWS_PER_B = J * O
M = 5
NI = J // M
NB = 4
G = 2


def _make_gather_kernel():
    mesh = plsc.VectorSubcoreMesh(core_axis_name="c", subcore_axis_name="s")

    scratch = (
        [pltpu.VMEM((J + 6, 128), jnp.int32)]
        + [pltpu.VMEM((M * BPW,), jnp.int32) for _ in range(NB)]
        + [pltpu.VMEM((M * BPW, D), jnp.float32) for _ in range(NB)]
        + [pltpu.VMEM((BPW, D), jnp.float32)]
        + [pltpu.SemaphoreType.DMA, pltpu.SemaphoreType.DMA,
           pltpu.SemaphoreType.DMA]
    )

    @functools.partial(
        pl.kernel,
        mesh=mesh,
        out_type=jax.ShapeDtypeStruct((J + 1, B, D), jnp.float32),
        scratch_types=scratch,
    )
    def gather_kernel(nodes_hbm, nop_hbm, skip_hbm, out_hbm, nop_v, *rest):
        idx_rings = rest[:NB]
        bufs = rest[NB:2 * NB]
        skip_v = rest[2 * NB]
        gsem, wsem, nsem = rest[2 * NB + 1:2 * NB + 4]

        wid = lax.axis_index("s") * NC + lax.axis_index("c")
        base_b = wid * BPW
        col = base_b % 128
        cb = (base_b // 128) * 128

        nh = [
            pltpu.async_copy(
                nop_hbm.at[pl.ds(q * 8, 8 if q < 6 else 2),
                           pl.ds(cb, 128)],
                nop_v.at[pl.ds(q * 8, 8 if q < 6 else 2)], nsem)
            for q in range(7)
        ]
        pltpu.sync_copy(skip_hbm, skip_v.at[pl.ds(0, 1)])
        chunks = [skip_v[0, pl.ds(c * L, L)] for c in range(D // L)]
        for r in range(1, BPW):
            for c in range(D // L):
                skip_v[r, pl.ds(c * L, L)] = chunks[c]
        pltpu.sync_copy(skip_v, out_hbm.at[0, pl.ds(base_b, BPW)])
        for h in nh:
            h.wait()

        lane = lax.broadcasted_iota(jnp.int32, (L,), 0)

        ghandles = {}
        whandles = {}
        for i in range(NI + G):
            if i < NI:
                p = i % NB
                if i >= NB:
                    for m in range(M):
                        whandles[(i - NB, m)].wait()
                for m in range(M):
                    j = i * M + m
                    for c in range(2):
                        k = c * L + lane
                        nop = nop_v[j, pl.ds(col + c * L, L)]
                        idx_rings[p][pl.ds(m * BPW + c * L, L)] = (
                            (base_b + k) * ROWS_PER_B + j * O + nop)
                ghandles[i] = pltpu.async_copy(
                    nodes_hbm.at[idx_rings[p]], bufs[p], gsem)
            if i >= G:
                k = i - G
                ghandles[k].wait()
                for m in range(M):
                    whandles[(k, m)] = pltpu.async_copy(
                        bufs[k % NB].at[pl.ds(m * BPW, BPW)],
                        out_hbm.at[k * M + m + 1, pl.ds(base_b, BPW)], wsem)
        for k in range(NI - NB, NI):
            for m in range(M):
                whandles[(k, m)].wait()

    return gather_kernel


_gather = _make_gather_kernel()


def kernel(nodes, op_scheduled, next_op, skip_token):
    table = nodes.reshape(B * J * O, D)
    nop = next_op.astype(jnp.int32).T
    skip = skip_token.reshape(1, D)
    out_t = _gather(table, nop, skip)
    return out_t.transpose(1, 0, 2)


# speedup vs baseline: 1.0049x; 1.0049x over previous
import functools

import jax
import jax.numpy as jnp
from jax import lax
from jax.experimental import pallas as pl
from jax.experimental.pallas import tpu as pltpu
from jax.experimental.pallas import tpu_sc as plsc

B, J, O, D = 1024, 50, 20, 128
NC, NS, L = 2, 16, 16
NW = NC * NS
BPW = B // NW
ROWS_PER_B = J * O
M = 5
NI = J // M
NB = 4
G = 2


def _make_gather_kernel():
    mesh = plsc.VectorSubcoreMesh(core_axis_name="c", subcore_axis_name="s")

    scratch = (
        [pltpu.VMEM((J + 6, 128), jnp.int32)]
        + [pltpu.VMEM((M * BPW,), jnp.int32) for _ in range(NB)]
        + [pltpu.VMEM((M * BPW, D), jnp.float32) for _ in range(NB)]
        + [pltpu.VMEM((BPW, D), jnp.float32)]
        + [pltpu.SemaphoreType.DMA, pltpu.SemaphoreType.DMA,
           pltpu.SemaphoreType.DMA]
    )

    @functools.partial(
        pl.kernel,
        mesh=mesh,
        out_type=jax.ShapeDtypeStruct((J + 1, B, D), jnp.float32),
        scratch_types=scratch,
    )
    def gather_kernel(nodes_hbm, nop_hbm, skip_hbm, out_hbm, nop_v, *rest):
        idx_rings = rest[:NB]
        bufs = rest[NB:2 * NB]
        skip_v = rest[2 * NB]
        gsem, wsem, nsem = rest[2 * NB + 1:2 * NB + 4]

        wid = lax.axis_index("s") * NC + lax.axis_index("c")
        base_b = wid * BPW
        col = base_b % 128
        cb = (base_b // 128) * 128

        nh = [
            pltpu.async_copy(
                nop_hbm.at[pl.ds(q * 8, 8 if q < 6 else 2),
                           pl.ds(cb, 128)],
                nop_v.at[pl.ds(q * 8, 8 if q < 6 else 2)], nsem)
            for q in range(7)
        ]
        pltpu.sync_copy(skip_hbm, skip_v.at[pl.ds(0, 1)])
        chunks = [skip_v[0, pl.ds(c * L, L)] for c in range(D // L)]

        def _rep(r, carry):
            for c in range(D // L):
                skip_v[r, pl.ds(c * L, L)] = chunks[c]
            return carry

        lax.fori_loop(1, BPW, _rep, 0)
        pltpu.sync_copy(skip_v, out_hbm.at[0, pl.ds(base_b, BPW)])
        for h in nh:
            h.wait()

        lane = lax.broadcasted_iota(jnp.int32, (L,), 0)

        ghandles = {}
        whandles = {}
        for i in range(NI + G):
            if i < NI:
                p = i % NB
                if i >= NB:
                    for m in range(M):
                        whandles[(i - NB, m)].wait()
                for m in range(M):
                    j = i * M + m
                    for c in range(2):
                        k = c * L + lane
                        nop = nop_v[j, pl.ds(col + c * L, L)]
                        idx_rings[p][pl.ds(m * BPW + c * L, L)] = (
                            (base_b + k) * ROWS_PER_B + j * O + nop)
                ghandles[i] = pltpu.async_copy(
                    nodes_hbm.at[idx_rings[p]], bufs[p], gsem)
            if i >= G:
                k = i - G
                ghandles[k].wait()
                for m in range(M):
                    whandles[(k, m)] = pltpu.async_copy(
                        bufs[k % NB].at[pl.ds(m * BPW, BPW)],
                        out_hbm.at[k * M + m + 1, pl.ds(base_b, BPW)], wsem)
        for k in range(NI - NB, NI):
            for m in range(M):
                whandles[(k, m)].wait()

    return gather_kernel


_gather = _make_gather_kernel()


def kernel(nodes, op_scheduled, next_op, skip_token):
    table = nodes.reshape(B * J * O, D)
    nop = next_op.astype(jnp.int32).T
    skip = skip_token.reshape(1, D)
    out_t = _gather(table, nop, skip)
    return out_t.transpose(1, 0, 2)


# speedup vs baseline: 1.0088x; 1.0039x over previous
import functools

import jax
import jax.numpy as jnp
from jax import lax
from jax.experimental import pallas as pl
from jax.experimental.pallas import tpu as pltpu
from jax.experimental.pallas import tpu_sc as plsc

B, J, O, D = 1024, 50, 20, 128
NC, NS, L = 2, 16, 16
NW = NC * NS
BPW = B // NW
ROWS_PER_B = J * O
M = 5
NI = J // M
NB = 5
G = 3


def _make_gather_kernel():
    mesh = plsc.VectorSubcoreMesh(core_axis_name="c", subcore_axis_name="s")

    scratch = (
        [pltpu.VMEM((J + 6, 128), jnp.int32)]
        + [pltpu.VMEM((M * BPW,), jnp.int32) for _ in range(NB)]
        + [pltpu.VMEM((M * BPW, D), jnp.float32) for _ in range(NB)]
        + [pltpu.VMEM((BPW, D), jnp.float32)]
        + [pltpu.SemaphoreType.DMA, pltpu.SemaphoreType.DMA,
           pltpu.SemaphoreType.DMA]
    )

    @functools.partial(
        pl.kernel,
        mesh=mesh,
        out_type=jax.ShapeDtypeStruct((J + 1, B, D), jnp.float32),
        scratch_types=scratch,
    )
    def gather_kernel(nodes_hbm, nop_hbm, skip_hbm, out_hbm, nop_v, *rest):
        idx_rings = rest[:NB]
        bufs = rest[NB:2 * NB]
        skip_v = rest[2 * NB]
        gsem, wsem, nsem = rest[2 * NB + 1:2 * NB + 4]

        wid = lax.axis_index("s") * NC + lax.axis_index("c")
        base_b = wid * BPW
        col = base_b % 128
        cb = (base_b // 128) * 128

        nh = [
            pltpu.async_copy(
                nop_hbm.at[pl.ds(q * 8, 8 if q < 6 else 2),
                           pl.ds(cb, 128)],
                nop_v.at[pl.ds(q * 8, 8 if q < 6 else 2)], nsem)
            for q in range(7)
        ]
        pltpu.sync_copy(skip_hbm, skip_v.at[pl.ds(0, 1)])
        chunks = [skip_v[0, pl.ds(c * L, L)] for c in range(D // L)]

        def _rep(r, carry):
            for c in range(D // L):
                skip_v[r, pl.ds(c * L, L)] = chunks[c]
            return carry

        lax.fori_loop(1, BPW, _rep, 0)
        pltpu.sync_copy(skip_v, out_hbm.at[0, pl.ds(base_b, BPW)])
        for h in nh:
            h.wait()

        lane = lax.broadcasted_iota(jnp.int32, (L,), 0)

        ghandles = {}
        whandles = {}
        for i in range(NI + G):
            if i < NI:
                p = i % NB
                if i >= NB:
                    for m in range(M):
                        whandles[(i - NB, m)].wait()
                for m in range(M):
                    j = i * M + m
                    for c in range(2):
                        k = c * L + lane
                        nop = nop_v[j, pl.ds(col + c * L, L)]
                        idx_rings[p][pl.ds(m * BPW + c * L, L)] = (
                            (base_b + k) * ROWS_PER_B + j * O + nop)
                ghandles[i] = pltpu.async_copy(
                    nodes_hbm.at[idx_rings[p]], bufs[p], gsem)
            if i >= G:
                k = i - G
                ghandles[k].wait()
                for m in range(M):
                    whandles[(k, m)] = pltpu.async_copy(
                        bufs[k % NB].at[pl.ds(m * BPW, BPW)],
                        out_hbm.at[k * M + m + 1, pl.ds(base_b, BPW)], wsem)
        for k in range(NI - NB, NI):
            for m in range(M):
                whandles[(k, m)].wait()

    return gather_kernel


_gather = _make_gather_kernel()


def kernel(nodes, op_scheduled, next_op, skip_token):
    table = nodes.reshape(B * J * O, D)
    nop = next_op.astype(jnp.int32).T
    skip = skip_token.reshape(1, D)
    out_t = _gather(table, nop, skip)
    return out_t.transpose(1, 0, 2)


# speedup vs baseline: 1.0176x; 1.0087x over previous
import functools

import jax
import jax.numpy as jnp
from jax import lax
from jax.experimental import pallas as pl
from jax.experimental.pallas import tpu as pltpu
from jax.experimental.pallas import tpu_sc as plsc

B, J, O, D = 1024, 50, 20, 128
NC, NS, L = 2, 16, 16
NW = NC * NS
BPW = B // NW
ROWS_PER_B = J * O
M = 5
NI = J // M
NB = 5
G = 3


def _make_gather_kernel():
    mesh = plsc.VectorSubcoreMesh(core_axis_name="c", subcore_axis_name="s")

    scratch = (
        [pltpu.VMEM((J + 6, 128), jnp.int32)]
        + [pltpu.VMEM((M * BPW,), jnp.int32) for _ in range(NB)]
        + [pltpu.VMEM((M * BPW, D), jnp.float32) for _ in range(NB)]
        + [pltpu.VMEM((BPW, D), jnp.float32)]
        + [pltpu.SemaphoreType.DMA, pltpu.SemaphoreType.DMA,
           pltpu.SemaphoreType.DMA]
    )

    @functools.partial(
        pl.kernel,
        mesh=mesh,
        out_type=jax.ShapeDtypeStruct((J + 1, B, D), jnp.float32),
        scratch_types=scratch,
    )
    def gather_kernel(nodes_hbm, nop_hbm, skip_hbm, out_hbm, nop_v, *rest):
        idx_rings = rest[:NB]
        bufs = rest[NB:2 * NB]
        skip_v = rest[2 * NB]
        gsem, wsem, nsem = rest[2 * NB + 1:2 * NB + 4]

        wid = lax.axis_index("s") * NC + lax.axis_index("c")
        base_b = wid * BPW
        col = base_b % 128
        cb = (base_b // 128) * 128

        nh = [
            pltpu.async_copy(nop_hbm.at[pl.ds(0, 48), pl.ds(cb, 128)],
                             nop_v.at[pl.ds(0, 48)], nsem),
            pltpu.async_copy(nop_hbm.at[pl.ds(48, 2), pl.ds(cb, 128)],
                             nop_v.at[pl.ds(48, 2)], nsem),
        ]
        hsk = pltpu.async_copy(skip_hbm, skip_v.at[pl.ds(0, 1)], wsem)
        for h in nh:
            h.wait()

        lane = lax.broadcasted_iota(jnp.int32, (L,), 0)

        ghandles = {}
        whandles = {}
        for i in range(NI + G):
            if i == G:
                hsk.wait()
                chunks = [skip_v[0, pl.ds(c * L, L)] for c in range(D // L)]

                def _rep(r, carry):
                    for c in range(D // L):
                        skip_v[r, pl.ds(c * L, L)] = chunks[c]
                    return carry

                lax.fori_loop(1, BPW, _rep, 0)
                pltpu.sync_copy(skip_v, out_hbm.at[0, pl.ds(base_b, BPW)])
            if i < NI:
                p = i % NB
                if i >= NB:
                    for m in range(M):
                        whandles[(i - NB, m)].wait()
                for m in range(M):
                    j = i * M + m
                    for c in range(2):
                        k = c * L + lane
                        nop = nop_v[j, pl.ds(col + c * L, L)]
                        idx_rings[p][pl.ds(m * BPW + c * L, L)] = (
                            (base_b + k) * ROWS_PER_B + j * O + nop)
                ghandles[i] = pltpu.async_copy(
                    nodes_hbm.at[idx_rings[p]], bufs[p], gsem)
            if i >= G:
                k = i - G
                ghandles[k].wait()
                for m in range(M):
                    whandles[(k, m)] = pltpu.async_copy(
                        bufs[k % NB].at[pl.ds(m * BPW, BPW)],
                        out_hbm.at[k * M + m + 1, pl.ds(base_b, BPW)], wsem)
        for k in range(NI - NB, NI):
            for m in range(M):
                whandles[(k, m)].wait()

    return gather_kernel


_gather = _make_gather_kernel()


def kernel(nodes, op_scheduled, next_op, skip_token):
    table = nodes.reshape(B * J * O, D)
    nop = next_op.astype(jnp.int32).T
    skip = skip_token.reshape(1, D)
    out_t = _gather(table, nop, skip)
    return out_t.transpose(1, 0, 2)


# speedup vs baseline: 1.0241x; 1.0064x over previous
import functools

import jax
import jax.numpy as jnp
from jax import lax
from jax.experimental import pallas as pl
from jax.experimental.pallas import tpu as pltpu
from jax.experimental.pallas import tpu_sc as plsc

B, J, O, D = 1024, 50, 20, 128
NC, NS, L = 2, 16, 16
NW = NC * NS
BPW = B // NW
ROWS_PER_B = J * O
M = 5
NI = J // M
NB = 5
G = 4


def _make_gather_kernel():
    mesh = plsc.VectorSubcoreMesh(core_axis_name="c", subcore_axis_name="s")

    scratch = (
        [pltpu.VMEM((J + 6, 128), jnp.int32)]
        + [pltpu.VMEM((M * BPW,), jnp.int32) for _ in range(NB)]
        + [pltpu.VMEM((M * BPW, D), jnp.float32) for _ in range(NB)]
        + [pltpu.VMEM((BPW, D), jnp.float32)]
        + [pltpu.SemaphoreType.DMA, pltpu.SemaphoreType.DMA,
           pltpu.SemaphoreType.DMA]
    )

    @functools.partial(
        pl.kernel,
        mesh=mesh,
        out_type=jax.ShapeDtypeStruct((J + 1, B, D), jnp.float32),
        scratch_types=scratch,
    )
    def gather_kernel(nodes_hbm, nop_hbm, skip_hbm, out_hbm, nop_v, *rest):
        idx_rings = rest[:NB]
        bufs = rest[NB:2 * NB]
        skip_v = rest[2 * NB]
        gsem, wsem, nsem = rest[2 * NB + 1:2 * NB + 4]

        wid = lax.axis_index("s") * NC + lax.axis_index("c")
        base_b = wid * BPW
        col = base_b % 128
        cb = (base_b // 128) * 128

        nh = [
            pltpu.async_copy(nop_hbm.at[pl.ds(0, 48), pl.ds(cb, 128)],
                             nop_v.at[pl.ds(0, 48)], nsem),
            pltpu.async_copy(nop_hbm.at[pl.ds(48, 2), pl.ds(cb, 128)],
                             nop_v.at[pl.ds(48, 2)], nsem),
        ]
        hsk = pltpu.async_copy(skip_hbm, skip_v.at[pl.ds(0, 1)], wsem)
        for h in nh:
            h.wait()

        lane = lax.broadcasted_iota(jnp.int32, (L,), 0)

        ghandles = {}
        whandles = {}
        for i in range(NI + G):
            if i == G:
                hsk.wait()
                chunks = [skip_v[0, pl.ds(c * L, L)] for c in range(D // L)]

                def _rep(r, carry):
                    for c in range(D // L):
                        skip_v[r, pl.ds(c * L, L)] = chunks[c]
                    return carry

                lax.fori_loop(1, BPW, _rep, 0)
                pltpu.sync_copy(skip_v, out_hbm.at[0, pl.ds(base_b, BPW)])
            if i < NI:
                p = i % NB
                if i >= NB:
                    for m in range(M):
                        whandles[(i - NB, m)].wait()
                for m in range(M):
                    j = i * M + m
                    for c in range(2):
                        k = c * L + lane
                        nop = nop_v[j, pl.ds(col + c * L, L)]
                        idx_rings[p][pl.ds(m * BPW + c * L, L)] = (
                            (base_b + k) * ROWS_PER_B + j * O + nop)
                ghandles[i] = pltpu.async_copy(
                    nodes_hbm.at[idx_rings[p]], bufs[p], gsem)
            if i >= G:
                k = i - G
                ghandles[k].wait()
                for m in range(M):
                    whandles[(k, m)] = pltpu.async_copy(
                        bufs[k % NB].at[pl.ds(m * BPW, BPW)],
                        out_hbm.at[k * M + m + 1, pl.ds(base_b, BPW)], wsem)
        for k in range(NI - NB, NI):
            for m in range(M):
                whandles[(k, m)].wait()

    return gather_kernel


_gather = _make_gather_kernel()


def kernel(nodes, op_scheduled, next_op, skip_token):
    table = nodes.reshape(B * J * O, D)
    nop = next_op.astype(jnp.int32).T
    skip = skip_token.reshape(1, D)
    out_t = _gather(table, nop, skip)
    return out_t.transpose(1, 0, 2)
